# baseline (speedup 1.0000x reference)
import functools

import jax
import jax.numpy as jnp
from jax.experimental import pallas as pl
from jax.experimental.pallas import tpu as pltpu

BEAM = 8
EOS = 2
EOS_T = 1.5
NSLOT = 8
W = 128
NEG = -1e30
IMAX = 2**31 - 1


def _step(x_ref, seq_ref, out_s_ref, out_c_ref, out_p_ref,
          tv_ref, ti_ref, m_ref, e_ref, *, vocab, vb, nv):
    b = pl.program_id(0)
    v = pl.program_id(1)

    x = x_ref[0]
    s = seq_ref[0]

    col = jax.lax.broadcasted_iota(jnp.int32, (BEAM, vb), 1) + v * vb
    x = jnp.where(col < vocab, x, NEG)

    rmax = jnp.max(x, axis=1, keepdims=True)

    @pl.when(v == 0)
    def _init():
        e_ref[...] = x[:, EOS:EOS + 1]
        m_ref[...] = rmax
        tv_ref[...] = jnp.full((NSLOT, BEAM, W), NEG, jnp.float32)
        ti_ref[...] = jnp.zeros((NSLOT, BEAM, W), jnp.int32)

    @pl.when(v > 0)
    def _acc():
        m_ref[...] = jnp.maximum(m_ref[...], rmax)

    xm = jnp.where(col == EOS, NEG, x)
    sc = xm + s

    def insert(val, idx):
        for k in range(NSLOT):
            tv = tv_ref[k]
            ti = ti_ref[k]
            gt = val > tv
            tv_ref[k] = jnp.where(gt, val, tv)
            ti_ref[k] = jnp.where(gt, idx, ti)
            val = jnp.where(gt, tv, val)
            idx = jnp.where(gt, ti, idx)

    for j in range(vb // W):
        insert(sc[:, j * W:(j + 1) * W], col[:, j * W:(j + 1) * W])

    @pl.when(v == nv - 1)
    def _finalize():
        keep = e_ref[...] > EOS_T * m_ref[...]
        ev = jnp.where(keep, e_ref[...] + s, NEG)
        lane = jax.lax.broadcasted_iota(jnp.int32, (BEAM, W), 1)
        insert(jnp.where(lane == 0, ev, NEG),
               jnp.full((BEAM, W), EOS, jnp.int32))

        cand = tv_ref[...].reshape(NSLOT * BEAM, W)
        ci = ti_ref[...].reshape(NSLOT * BEAM, W)
        row = jax.lax.broadcasted_iota(jnp.int32, (NSLOT * BEAM, W), 0) % BEAM
        flat = row * vocab + ci

        k8 = jax.lax.broadcasted_iota(jnp.int32, (1, 1, BEAM), 2)
        out_s = jnp.zeros((1, 1, BEAM), jnp.float32)
        out_f = jnp.zeros((1, 1, BEAM), jnp.int32)
        for k in range(BEAM):
            m = jnp.max(cand)
            hit = cand == m
            pf = jnp.min(jnp.where(hit, flat, IMAX))
            out_s = jnp.where(k8 == k, m, out_s)
            out_f = jnp.where(k8 == k, pf, out_f)
            cand = jnp.where(hit & (flat == pf), NEG, cand)

        out_s_ref[...] = out_s
        out_c_ref[...] = out_f % vocab
        out_p_ref[...] = out_f // vocab + b * BEAM


def kernel(log_probs, sequence_scores):
    rows, vocab = log_probs.shape
    batch = rows // BEAM
    vb = 2048 if vocab >= 2048 else W * pl.cdiv(vocab, W)
    nv = pl.cdiv(vocab, vb)

    seq3 = sequence_scores.reshape(batch, BEAM, 1)
    lp3 = log_probs.reshape(batch, BEAM, vocab)

    out_shape = (
        jax.ShapeDtypeStruct((batch, 1, BEAM), jnp.float32),
        jax.ShapeDtypeStruct((batch, 1, BEAM), jnp.int32),
        jax.ShapeDtypeStruct((batch, 1, BEAM), jnp.int32),
    )
    grid = (batch, nv)
    out_spec = pl.BlockSpec((1, 1, BEAM), lambda b, v: (b, 0, 0))
    scores, cands, preds = pl.pallas_call(
        functools.partial(_step, vocab=vocab, vb=vb, nv=nv),
        grid=grid,
        in_specs=[
            pl.BlockSpec((1, BEAM, vb), lambda b, v: (b, 0, v)),
            pl.BlockSpec((1, BEAM, 1), lambda b, v: (b, 0, 0)),
        ],
        out_specs=(out_spec, out_spec, out_spec),
        out_shape=out_shape,
        scratch_shapes=[
            pltpu.VMEM((NSLOT, BEAM, W), jnp.float32),
            pltpu.VMEM((NSLOT, BEAM, W), jnp.int32),
            pltpu.VMEM((BEAM, 1), jnp.float32),
            pltpu.VMEM((BEAM, 1), jnp.float32),
        ],
        compiler_params=pltpu.CompilerParams(
            dimension_semantics=("arbitrary", "arbitrary"),
        ),
    )(lp3, seq3)

    return (scores.reshape(batch, BEAM),
            cands.reshape(batch, BEAM),
            preds.reshape(batch, BEAM))
